# baseline (speedup 1.0000x reference)
import jax
import jax.numpy as jnp
from jax import lax
from jax.experimental import pallas as pl
from jax.experimental.pallas import tpu as pltpu
from jax.experimental.pallas import tpu_sc as plsc

N, DEG, D = 10000, 16, 256
L = 16
CHUNKS = D // L
NB = 8
N_SC = 4000
N_TC = N - N_SC
BN = 400
NUM_BATCHES = N_SC // NB
NW = 32

_mesh = plsc.VectorSubcoreMesh(core_axis_name="c", subcore_axis_name="s")


def _tree_reduce_chunk(xslot, ovslot, it):
    n = it // CHUNKS
    col = (it % CHUNKS) * L
    vs = [xslot[n, i, pl.ds(col, L)] for i in range(DEG)]
    ms = vs
    while len(vs) > 1:
        vs = [vs[k] + vs[k + 1] for k in range(0, len(vs), 2)]
        ms = [jnp.maximum(ms[k], ms[k + 1]) for k in range(0, len(ms), 2)]
    ovslot[n, pl.ds(col, L)] = vs[0] * (1.0 / DEG)
    ovslot[n, pl.ds(D + col, L)] = ms[0]


def _kernel_body(x_hbm, out_hbm, xv0, xv1, ov0, ov1, is0, is1, os0, os1):
    wid = lax.axis_index("s") * 2 + lax.axis_index("c")
    nbat = (NUM_BATCHES - wid + NW - 1) // NW
    npairs = nbat // 2

    def base(t):
        return (wid + t * NW) * NB

    def start_in(t, xslot, sem):
        pltpu.async_copy(x_hbm.at[pl.ds(base(t), NB)], xslot, sem)

    def wait_in(t, xslot, sem):
        pltpu.make_async_copy(x_hbm.at[pl.ds(base(t), NB)], xslot, sem).wait()

    def start_out(t, ovslot, sem):
        pltpu.async_copy(ovslot, out_hbm.at[pl.ds(base(t), NB)], sem)

    def wait_out(t, ovslot, sem):
        pltpu.make_async_copy(ovslot, out_hbm.at[pl.ds(base(t), NB)], sem).wait()

    start_in(0, xv0, is0)
    start_in(1, xv1, is1)

    def pair_body(p, _):
        for k, (xs, ovs, isem, osem) in enumerate(
            ((xv0, ov0, is0, os0), (xv1, ov1, is1, os1))
        ):
            t = 2 * p + k
            wait_in(t, xs, isem)

            @pl.when(p > 0)
            def _():
                wait_out(t - 2, ovs, osem)

            @plsc.parallel_loop(0, NB * CHUNKS, 1, unroll=2)
            def _(it):
                _tree_reduce_chunk(xs, ovs, it)

            @pl.when(t + 2 < nbat)
            def _():
                start_in(t + 2, xs, isem)

            start_out(t, ovs, osem)
        return 0

    lax.fori_loop(0, npairs, pair_body, 0)

    @pl.when(nbat % 2 == 1)
    def _():
        t = nbat - 1
        wait_in(t, xv0, is0)
        wait_out(t - 2, ov0, os0)

        @plsc.parallel_loop(0, NB * CHUNKS, 1, unroll=2)
        def _(it):
            _tree_reduce_chunk(xv0, ov0, it)

        start_out(t, ov0, os0)
        wait_out(t, ov0, os0)
        wait_out(nbat - 2, ov1, os1)

    @pl.when(nbat % 2 == 0)
    def _():
        wait_out(nbat - 2, ov0, os0)
        wait_out(nbat - 1, ov1, os1)


_mean_max_sc = pl.kernel(
    _kernel_body,
    out_type=jax.ShapeDtypeStruct((N_SC, 2 * D), jnp.float32),
    mesh=_mesh,
    scratch_types=[
        pltpu.VMEM((NB, DEG, D), jnp.float32),
        pltpu.VMEM((NB, DEG, D), jnp.float32),
        pltpu.VMEM((NB, 2 * D), jnp.float32),
        pltpu.VMEM((NB, 2 * D), jnp.float32),
        pltpu.SemaphoreType.DMA,
        pltpu.SemaphoreType.DMA,
        pltpu.SemaphoreType.DMA,
        pltpu.SemaphoreType.DMA,
    ],
)


def _tc_body(x_ref, o_ref):
    x = x_ref[...]
    o_ref[:, :D] = jnp.mean(x, axis=1)
    o_ref[:, D:] = jnp.max(x, axis=1)


_mean_max_tc = pl.pallas_call(
    _tc_body,
    out_shape=jax.ShapeDtypeStruct((N_TC, 2 * D), jnp.float32),
    grid=(N_TC // BN,),
    in_specs=[pl.BlockSpec((BN, DEG, D), lambda i: (i + N_SC // BN, 0, 0))],
    out_specs=pl.BlockSpec((BN, 2 * D), lambda i: (i, 0)),
)


def kernel(feat_dist):
    sc_out = _mean_max_sc(feat_dist)
    tc_out = _mean_max_tc(feat_dist)
    return jnp.concatenate([sc_out, tc_out], axis=0)


# speedup vs baseline: 1.1002x; 1.1002x over previous
import jax
import jax.numpy as jnp
from jax import lax
from jax.experimental import pallas as pl
from jax.experimental.pallas import tpu as pltpu
from jax.experimental.pallas import tpu_sc as plsc

N, DEG, D = 10000, 16, 256
L = 16
CHUNKS = D // L
NB = 8
NUM_BATCHES = N // NB
NW = 32

_mesh = plsc.VectorSubcoreMesh(core_axis_name="c", subcore_axis_name="s")


def _tree_reduce_chunk(xslot, ovslot, it):
    n = it // CHUNKS
    col = (it % CHUNKS) * L
    vs = [xslot[n, i, pl.ds(col, L)] for i in range(DEG)]
    ms = vs
    while len(vs) > 1:
        vs = [vs[k] + vs[k + 1] for k in range(0, len(vs), 2)]
        ms = [jnp.maximum(ms[k], ms[k + 1]) for k in range(0, len(ms), 2)]
    ovslot[n, pl.ds(col, L)] = vs[0] * (1.0 / DEG)
    ovslot[n, pl.ds(D + col, L)] = ms[0]


def _kernel_body(x_hbm, out_hbm, xv0, xv1, xv2, ov0, ov1, ov2,
                 is0, is1, is2, os0, os1, os2):
    wid = lax.axis_index("s") * 2 + lax.axis_index("c")
    nbat = (NUM_BATCHES - wid + NW - 1) // NW
    ntrip = nbat // 3
    rem = nbat - 3 * ntrip

    slots = ((xv0, ov0, is0, os0), (xv1, ov1, is1, os1), (xv2, ov2, is2, os2))

    def base(t):
        return (wid + t * NW) * NB

    def start_in(t, xslot, sem):
        pltpu.async_copy(x_hbm.at[pl.ds(base(t), NB)], xslot, sem)

    def wait_in(xslot, sem):
        pltpu.make_async_copy(x_hbm.at[pl.ds(0, NB)], xslot, sem).wait()

    def start_out(t, ovslot, sem):
        pltpu.async_copy(ovslot, out_hbm.at[pl.ds(base(t), NB)], sem)

    def wait_out(ovslot, sem):
        pltpu.make_async_copy(ovslot, out_hbm.at[pl.ds(0, NB)], sem).wait()

    def process(t, p, xs, ovs, isem, osem):
        wait_in(xs, isem)

        @pl.when(p > 0)
        def _():
            wait_out(ovs, osem)

        @plsc.parallel_loop(0, NB * CHUNKS, 1, unroll=2)
        def _(it):
            _tree_reduce_chunk(xs, ovs, it)

        @pl.when(t + 3 < nbat)
        def _():
            start_in(t + 3, xs, isem)

        start_out(t, ovs, osem)

    start_in(0, xv0, is0)
    start_in(1, xv1, is1)
    start_in(2, xv2, is2)

    def trip_body(p, _):
        for k, (xs, ovs, isem, osem) in enumerate(slots):
            process(3 * p + k, p, xs, ovs, isem, osem)
        return 0

    lax.fori_loop(0, ntrip, trip_body, 0)

    @pl.when(rem >= 1)
    def _():
        process(3 * ntrip, ntrip, xv0, ov0, is0, os0)

    @pl.when(rem >= 2)
    def _():
        process(3 * ntrip + 1, ntrip, xv1, ov1, is1, os1)

    wait_out(ov0, os0)
    wait_out(ov1, os1)
    wait_out(ov2, os2)


_mean_max = pl.kernel(
    _kernel_body,
    out_type=jax.ShapeDtypeStruct((N, 2 * D), jnp.float32),
    mesh=_mesh,
    scratch_types=[
        pltpu.VMEM((NB, DEG, D), jnp.float32),
        pltpu.VMEM((NB, DEG, D), jnp.float32),
        pltpu.VMEM((NB, DEG, D), jnp.float32),
        pltpu.VMEM((NB, 2 * D), jnp.float32),
        pltpu.VMEM((NB, 2 * D), jnp.float32),
        pltpu.VMEM((NB, 2 * D), jnp.float32),
        pltpu.SemaphoreType.DMA,
        pltpu.SemaphoreType.DMA,
        pltpu.SemaphoreType.DMA,
        pltpu.SemaphoreType.DMA,
        pltpu.SemaphoreType.DMA,
        pltpu.SemaphoreType.DMA,
    ],
)


def kernel(feat_dist):
    return _mean_max(feat_dist)


# speedup vs baseline: 1.1011x; 1.0009x over previous
import functools

import jax
import jax.numpy as jnp
from jax import lax
from jax.experimental import pallas as pl
from jax.experimental.pallas import tpu as pltpu
from jax.experimental.pallas import tpu_sc as plsc

N, DEG, D = 10000, 16, 256
L = 16
CHUNKS = D // L
NB = 8
NUM_BATCHES = N // NB
NW = 32


def _tree_reduce_chunk(xslot, ovslot, it):
    n = it // CHUNKS
    col = (it % CHUNKS) * L
    vs = [xslot[n, i, pl.ds(col, L)] for i in range(DEG)]
    ms = vs
    while len(vs) > 1:
        vs = [vs[k] + vs[k + 1] for k in range(0, len(vs), 2)]
        ms = [jnp.maximum(ms[k], ms[k + 1]) for k in range(0, len(ms), 2)]
    ovslot[n, pl.ds(col, L)] = vs[0] * (1.0 / DEG)
    ovslot[n, pl.ds(D + col, L)] = ms[0]


def _kernel_body(x_hbm, out_hbm, xv0, xv1, xv2, ov0, ov1, ov2,
                 is0, is1, is2, os0, os1, os2):
    wid = lax.axis_index("s") * 2 + lax.axis_index("c")
    nbat = (NUM_BATCHES - wid + NW - 1) // NW
    ntrip = nbat // 3
    rem = nbat - 3 * ntrip

    slots = ((xv0, ov0, is0, os0), (xv1, ov1, is1, os1), (xv2, ov2, is2, os2))

    def base(t):
        return (wid + t * NW) * NB

    def start_in(t, xslot, sem):
        pltpu.async_copy(x_hbm.at[pl.ds(base(t), NB)], xslot, sem)

    def wait_in(xslot, sem):
        pltpu.make_async_copy(x_hbm.at[pl.ds(0, NB)], xslot, sem).wait()

    def start_out(t, ovslot, sem):
        pltpu.async_copy(ovslot, out_hbm.at[pl.ds(base(t), NB)], sem)

    def wait_out(ovslot, sem):
        pltpu.make_async_copy(ovslot, out_hbm.at[pl.ds(0, NB)], sem).wait()

    def process(t, p, xs, ovs, isem, osem):
        wait_in(xs, isem)

        @pl.when(p > 0)
        def _():
            wait_out(ovs, osem)

        @plsc.parallel_loop(0, NB * CHUNKS, 1, unroll=2)
        def _(it):
            _tree_reduce_chunk(xs, ovs, it)

        @pl.when(t + 3 < nbat)
        def _():
            start_in(t + 3, xs, isem)

        start_out(t, ovs, osem)

    start_in(0, xv0, is0)
    start_in(1, xv1, is1)
    start_in(2, xv2, is2)

    def trip_body(p, _):
        for k, (xs, ovs, isem, osem) in enumerate(slots):
            process(3 * p + k, p, xs, ovs, isem, osem)
        return 0

    lax.fori_loop(0, ntrip, trip_body, 0)

    @pl.when(rem >= 1)
    def _():
        process(3 * ntrip, ntrip, xv0, ov0, is0, os0)

    @pl.when(rem >= 2)
    def _():
        process(3 * ntrip + 1, ntrip, xv1, ov1, is1, os1)

    wait_out(ov0, os0)
    wait_out(ov1, os1)
    wait_out(ov2, os2)


@functools.cache
def _build():
    mesh = plsc.VectorSubcoreMesh(core_axis_name="c", subcore_axis_name="s")
    return pl.kernel(
        _kernel_body,
        out_type=jax.ShapeDtypeStruct((N, 2 * D), jnp.float32),
        mesh=mesh,
        scratch_types=[
            pltpu.VMEM((NB, DEG, D), jnp.float32),
            pltpu.VMEM((NB, DEG, D), jnp.float32),
            pltpu.VMEM((NB, DEG, D), jnp.float32),
            pltpu.VMEM((NB, 2 * D), jnp.float32),
            pltpu.VMEM((NB, 2 * D), jnp.float32),
            pltpu.VMEM((NB, 2 * D), jnp.float32),
            pltpu.SemaphoreType.DMA,
            pltpu.SemaphoreType.DMA,
            pltpu.SemaphoreType.DMA,
            pltpu.SemaphoreType.DMA,
            pltpu.SemaphoreType.DMA,
            pltpu.SemaphoreType.DMA,
        ],
    )


def kernel(feat_dist):
    return _build()(feat_dist)
